# Initial kernel scaffold; baseline (speedup 1.0000x reference)
import jax
import jax.numpy as jnp
from jax.experimental import pallas as pl


def kernel(x, W1, b1, Wr, br, We, be, W2, b2):
    raise NotImplementedError("write your pallas kernel here")


# speedup vs baseline: 4.4841x; 4.4841x over previous
import functools

import jax
import jax.numpy as jnp
from jax import lax
from jax.experimental import pallas as pl
from jax.experimental.pallas import tpu as pltpu

C1 = 384
C2 = 384
C = 192
E = 4
TOPK = 2
H = 56
W = 56
HP = H + 2
WP = W + 2
NP = HP * WP
NI = (H - 1) * WP + W
OFF0 = WP + 1
STARTS = tuple(i * WP + j for i in range(3) for j in range(3))


def _silu(v):
    return v * jax.nn.sigmoid(v)


def _cv1_router_body(xp_ref, W1_ref, b1_ref, Wr_ref, br_ref,
                     t_ref, idx_ref, wts_ref):
    xb = xp_ref[0]
    t = jnp.dot(W1_ref[...], xb, preferred_element_type=jnp.float32) + b1_ref[...]
    t = _silu(t)
    col = lax.broadcasted_iota(jnp.int32, (1, NP), 1)
    rr = col // WP
    cc = col % WP
    valid = ((rr >= 1) & (rr <= H) & (cc >= 1) & (cc <= W))
    t = jnp.where(valid, t, 0.0)
    t_ref[0] = t
    y1 = t[C:]
    pooled = jnp.sum(y1, axis=1, keepdims=True) * jnp.float32(1.0 / (H * W))
    logits = jnp.dot(Wr_ref[...], pooled, preferred_element_type=jnp.float32) + br_ref[...]
    m = jnp.max(logits)
    ex = jnp.exp(logits - m)
    p = ex / jnp.sum(ex)
    io = lax.broadcasted_iota(jnp.int32, (E, 1), 0)
    p1 = jnp.max(p)
    i1 = jnp.min(jnp.where(p >= p1, io, E))
    pm = jnp.where(io == i1, -1.0, p)
    p2 = jnp.max(pm)
    i2 = jnp.min(jnp.where(pm >= p2, io, E))
    s = p1 + p2
    k_iota = lax.broadcasted_iota(jnp.int32, (1, 1, TOPK), 2)
    idx_ref[...] = jnp.where(k_iota == 0, i1, i2)
    wts_ref[...] = jnp.where(k_iota == 0, p1 / s, p2 / s)


def _expert_cv2_body(idx_sref, wts_sref, t_ref, Wm1_ref, Wm2_ref,
                     be1_ref, be2_ref, W2_ref, b2_ref, out_ref):
    b = pl.program_id(0)
    w1 = wts_sref[2 * b]
    w2 = wts_sref[2 * b + 1]
    acc = None
    for s in range(9):
        wcat = jnp.concatenate([Wm1_ref[0, s], Wm2_ref[0, s]], axis=0)
        xs = t_ref[0, C:, STARTS[s]:STARTS[s] + NI]
        d = jnp.dot(wcat, xs, preferred_element_type=jnp.float32)
        acc = d if acc is None else acc + d
    e1 = _silu(acc[:C] + be1_ref[0])
    e2 = _silu(acc[C:] + be2_ref[0])
    eo = w1 * e1 + w2 * e2
    y0i = t_ref[0, :C, OFF0:OFF0 + NI]
    y1i = t_ref[0, C:, OFF0:OFF0 + NI]
    o = (jnp.dot(W2_ref[:, :C], y0i, preferred_element_type=jnp.float32)
         + jnp.dot(W2_ref[:, C:2 * C], y1i, preferred_element_type=jnp.float32)
         + jnp.dot(W2_ref[:, 2 * C:], eo, preferred_element_type=jnp.float32)
         + b2_ref[...])
    out_ref[0] = _silu(o)


def kernel(x, W1, b1, Wr, br, We, be, W2, b2):
    B = x.shape[0]
    xp = jnp.pad(x, ((0, 0), (0, 0), (1, 1), (1, 1))).reshape(B, C1, NP)
    W1r = W1.reshape(2 * C, C1)
    b1c = b1.reshape(2 * C, 1)
    brc = br.reshape(E, 1)
    Wm = We.transpose(0, 3, 4, 1, 2).reshape(E, 9, C, C)
    bec = be.reshape(E, C, 1)
    W2r = W2.reshape(C2, (2 + 1) * C)
    b2c = b2.reshape(C2, 1)

    t, idx, wts = pl.pallas_call(
        _cv1_router_body,
        grid=(B,),
        in_specs=[
            pl.BlockSpec((1, C1, NP), lambda b: (b, 0, 0)),
            pl.BlockSpec((2 * C, C1), lambda b: (0, 0)),
            pl.BlockSpec((2 * C, 1), lambda b: (0, 0)),
            pl.BlockSpec((E, C), lambda b: (0, 0)),
            pl.BlockSpec((E, 1), lambda b: (0, 0)),
        ],
        out_specs=[
            pl.BlockSpec((1, 2 * C, NP), lambda b: (b, 0, 0)),
            pl.BlockSpec((1, 1, TOPK), lambda b: (b, 0, 0)),
            pl.BlockSpec((1, 1, TOPK), lambda b: (b, 0, 0)),
        ],
        out_shape=[
            jax.ShapeDtypeStruct((B, 2 * C, NP), jnp.float32),
            jax.ShapeDtypeStruct((B, 1, TOPK), jnp.int32),
            jax.ShapeDtypeStruct((B, 1, TOPK), jnp.float32),
        ],
    )(xp, W1r, b1c, Wr, brc)

    idx_flat = idx.reshape(B * TOPK)
    wts_flat = wts.reshape(B * TOPK)

    grid_spec = pltpu.PrefetchScalarGridSpec(
        num_scalar_prefetch=2,
        grid=(B,),
        in_specs=[
            pl.BlockSpec((1, 2 * C, NP), lambda b, I, Ww: (b, 0, 0)),
            pl.BlockSpec((1, 9, C, C), lambda b, I, Ww: (I[2 * b], 0, 0, 0)),
            pl.BlockSpec((1, 9, C, C), lambda b, I, Ww: (I[2 * b + 1], 0, 0, 0)),
            pl.BlockSpec((1, C, 1), lambda b, I, Ww: (I[2 * b], 0, 0)),
            pl.BlockSpec((1, C, 1), lambda b, I, Ww: (I[2 * b + 1], 0, 0)),
            pl.BlockSpec((C2, 3 * C), lambda b, I, Ww: (0, 0)),
            pl.BlockSpec((C2, 1), lambda b, I, Ww: (0, 0)),
        ],
        out_specs=pl.BlockSpec((1, C2, NI), lambda b, I, Ww: (b, 0, 0)),
    )
    out_i = pl.pallas_call(
        _expert_cv2_body,
        grid_spec=grid_spec,
        out_shape=jax.ShapeDtypeStruct((B, C2, NI), jnp.float32),
    )(idx_flat, wts_flat, t, Wm, Wm, bec, bec, W2r, b2c)

    out = jnp.pad(out_i, ((0, 0), (0, 0), (OFF0, NP - NI - OFF0)))
    out = out.reshape(B, C2, HP, WP)[:, :, 1:1 + H, 1:1 + W]
    return out


# speedup vs baseline: 7.2167x; 1.6094x over previous
import functools

import jax
import jax.numpy as jnp
from jax import lax
from jax.experimental import pallas as pl
from jax.experimental.pallas import tpu as pltpu

C1 = 384
C2 = 384
C = 192
E = 4
TOPK = 2
H = 56
W = 56
HP = H + 2
WP = W + 2
NP = HP * WP
NI = (H - 1) * WP + W
OFF0 = WP + 1
STARTS = tuple(i * WP + j for i in range(3) for j in range(3))


def _silu(v):
    return v * jax.nn.sigmoid(v)


def _cv1_router_body(x_ref, W1_ref, b1_ref, Wr_ref, br_ref,
                     t_ref, idx_ref, wts_ref):
    xb = x_ref[0]
    t = jnp.dot(W1_ref[...], xb, preferred_element_type=jnp.float32) + b1_ref[...]
    t = _silu(t)
    t_ref[0] = jnp.zeros((2 * C, NP), jnp.float32)
    for h in range(H):
        t_ref[0, :, (h + 1) * WP + 1:(h + 1) * WP + 1 + W] = t[:, h * W:(h + 1) * W]
    y1 = t[C:]
    pooled = jnp.sum(y1, axis=1, keepdims=True) * jnp.float32(1.0 / (H * W))
    logits = jnp.dot(Wr_ref[...], pooled, preferred_element_type=jnp.float32) + br_ref[...]
    m = jnp.max(logits)
    ex = jnp.exp(logits - m)
    p = ex / jnp.sum(ex)
    io = lax.broadcasted_iota(jnp.int32, (E, 1), 0)
    p1 = jnp.max(p)
    i1 = jnp.min(jnp.where(p >= p1, io, E))
    pm = jnp.where(io == i1, -1.0, p)
    p2 = jnp.max(pm)
    i2 = jnp.min(jnp.where(pm >= p2, io, E))
    s = p1 + p2
    k_iota = lax.broadcasted_iota(jnp.int32, (1, 1, TOPK), 2)
    idx_ref[...] = jnp.where(k_iota == 0, i1, i2)
    wts_ref[...] = jnp.where(k_iota == 0, p1 / s, p2 / s)


def _expert_cv2_body(idx_sref, wts_sref, t_ref, Wm1_ref, Wm2_ref,
                     be1_ref, be2_ref, W2_ref, b2_ref, out_ref):
    b = pl.program_id(0)
    w1 = wts_sref[2 * b]
    w2 = wts_sref[2 * b + 1]
    acc = None
    for s in range(9):
        wcat = jnp.concatenate([Wm1_ref[0, s], Wm2_ref[0, s]], axis=0)
        xs = t_ref[0, C:, STARTS[s]:STARTS[s] + NI]
        d = jnp.dot(wcat, xs, preferred_element_type=jnp.float32)
        acc = d if acc is None else acc + d
    e1 = _silu(acc[:C] + be1_ref[0])
    e2 = _silu(acc[C:] + be2_ref[0])
    eo = w1 * e1 + w2 * e2
    y0i = t_ref[0, :C, OFF0:OFF0 + NI]
    y1i = t_ref[0, C:, OFF0:OFF0 + NI]
    o = (jnp.dot(W2_ref[:, :C], y0i, preferred_element_type=jnp.float32)
         + jnp.dot(W2_ref[:, C:2 * C], y1i, preferred_element_type=jnp.float32)
         + jnp.dot(W2_ref[:, 2 * C:], eo, preferred_element_type=jnp.float32)
         + b2_ref[...])
    o = _silu(o)
    for h in range(H):
        out_ref[0, :, h * W:(h + 1) * W] = o[:, h * WP:h * WP + W]


def kernel(x, W1, b1, Wr, br, We, be, W2, b2):
    B = x.shape[0]
    xf = x.reshape(B, C1, H * W)
    W1r = W1.reshape(2 * C, C1)
    b1c = b1.reshape(2 * C, 1)
    brc = br.reshape(E, 1)
    Wm = We.transpose(0, 3, 4, 1, 2).reshape(E, 9, C, C)
    bec = be.reshape(E, C, 1)
    W2r = W2.reshape(C2, (2 + 1) * C)
    b2c = b2.reshape(C2, 1)

    t, idx, wts = pl.pallas_call(
        _cv1_router_body,
        grid=(B,),
        in_specs=[
            pl.BlockSpec((1, C1, H * W), lambda b: (b, 0, 0)),
            pl.BlockSpec((2 * C, C1), lambda b: (0, 0)),
            pl.BlockSpec((2 * C, 1), lambda b: (0, 0)),
            pl.BlockSpec((E, C), lambda b: (0, 0)),
            pl.BlockSpec((E, 1), lambda b: (0, 0)),
        ],
        out_specs=[
            pl.BlockSpec((1, 2 * C, NP), lambda b: (b, 0, 0)),
            pl.BlockSpec((1, 1, TOPK), lambda b: (b, 0, 0)),
            pl.BlockSpec((1, 1, TOPK), lambda b: (b, 0, 0)),
        ],
        out_shape=[
            jax.ShapeDtypeStruct((B, 2 * C, NP), jnp.float32),
            jax.ShapeDtypeStruct((B, 1, TOPK), jnp.int32),
            jax.ShapeDtypeStruct((B, 1, TOPK), jnp.float32),
        ],
    )(xf, W1r, b1c, Wr, brc)

    idx_flat = idx.reshape(B * TOPK)
    wts_flat = wts.reshape(B * TOPK)

    grid_spec = pltpu.PrefetchScalarGridSpec(
        num_scalar_prefetch=2,
        grid=(B,),
        in_specs=[
            pl.BlockSpec((1, 2 * C, NP), lambda b, I, Ww: (b, 0, 0)),
            pl.BlockSpec((1, 9, C, C), lambda b, I, Ww: (I[2 * b], 0, 0, 0)),
            pl.BlockSpec((1, 9, C, C), lambda b, I, Ww: (I[2 * b + 1], 0, 0, 0)),
            pl.BlockSpec((1, C, 1), lambda b, I, Ww: (I[2 * b], 0, 0)),
            pl.BlockSpec((1, C, 1), lambda b, I, Ww: (I[2 * b + 1], 0, 0)),
            pl.BlockSpec((C2, 3 * C), lambda b, I, Ww: (0, 0)),
            pl.BlockSpec((C2, 1), lambda b, I, Ww: (0, 0)),
        ],
        out_specs=pl.BlockSpec((1, C2, H * W), lambda b, I, Ww: (b, 0, 0)),
    )
    out_f = pl.pallas_call(
        _expert_cv2_body,
        grid_spec=grid_spec,
        out_shape=jax.ShapeDtypeStruct((B, C2, H * W), jnp.float32),
    )(idx_flat, wts_flat, t, Wm, Wm, bec, bec, W2r, b2c)

    return out_f.reshape(B, C2, H, W)


# speedup vs baseline: 7.9319x; 1.0991x over previous
import jax
import jax.numpy as jnp
from jax import lax
from jax.experimental import pallas as pl
from jax.experimental.pallas import tpu as pltpu

C1 = 384
C2 = 384
C = 192
E = 4
TOPK = 2
H = 56
W = 56
HP = H + 2
WP = W + 2
NP = HP * WP
NI = (H - 1) * WP + W
OFF0 = WP + 1
STARTS = tuple(i * WP + j for i in range(3) for j in range(3))


def _silu(v):
    return v * jax.nn.sigmoid(v)


def _fused_body(x_ref, W1_ref, b1_ref, Wr_ref, br_ref, Wm_ref, be_ref,
                W2_ref, b2_ref, out_ref, y1p_ref):
    xb = x_ref[0]
    t = jnp.dot(W1_ref[...], xb, preferred_element_type=jnp.float32) + b1_ref[...]
    t = _silu(t)
    y0 = t[:C]
    y1 = t[C:]
    y1p_ref[...] = jnp.zeros((C, NP), jnp.float32)
    for h in range(H):
        y1p_ref[:, (h + 1) * WP + 1:(h + 1) * WP + 1 + W] = y1[:, h * W:(h + 1) * W]
    pooled = jnp.sum(y1, axis=1, keepdims=True) * jnp.float32(1.0 / (H * W))
    logits = jnp.dot(Wr_ref[...], pooled, preferred_element_type=jnp.float32) + br_ref[...]
    m = jnp.max(logits)
    ex = jnp.exp(logits - m)
    p = ex / jnp.sum(ex)
    io = lax.broadcasted_iota(jnp.int32, (E, 1), 0)
    p1 = jnp.max(p)
    i1 = jnp.min(jnp.where(p >= p1, io, E))
    pm = jnp.where(io == i1, -1.0, p)
    p2 = jnp.max(pm)
    i2 = jnp.min(jnp.where(pm >= p2, io, E))
    s = p1 + p2
    w1 = p1 / s
    w2 = p2 / s
    acc = None
    for sidx in range(9):
        wcat = jnp.concatenate([Wm_ref[i1, sidx], Wm_ref[i2, sidx]], axis=0)
        xs = y1p_ref[:, STARTS[sidx]:STARTS[sidx] + NI]
        d = jnp.dot(wcat, xs, preferred_element_type=jnp.float32)
        acc = d if acc is None else acc + d
    e1 = _silu(acc[:C] + be_ref[i1])
    e2 = _silu(acc[C:] + be_ref[i2])
    eo = w1 * e1 + w2 * e2
    eod = jnp.concatenate([eo[:, h * WP:h * WP + W] for h in range(H)], axis=1)
    o = (jnp.dot(W2_ref[:, :C], y0, preferred_element_type=jnp.float32)
         + jnp.dot(W2_ref[:, C:2 * C], y1, preferred_element_type=jnp.float32)
         + jnp.dot(W2_ref[:, 2 * C:], eod, preferred_element_type=jnp.float32)
         + b2_ref[...])
    out_ref[0] = _silu(o)


def kernel(x, W1, b1, Wr, br, We, be, W2, b2):
    B = x.shape[0]
    xf = x.reshape(B, C1, H * W)
    W1r = W1.reshape(2 * C, C1)
    b1c = b1.reshape(2 * C, 1)
    brc = br.reshape(E, 1)
    Wm = We.transpose(0, 3, 4, 1, 2).reshape(E, 9, C, C)
    bec = be.reshape(E, C, 1)
    W2r = W2.reshape(C2, (2 + 1) * C)
    b2c = b2.reshape(C2, 1)

    out_f = pl.pallas_call(
        _fused_body,
        grid=(B,),
        in_specs=[
            pl.BlockSpec((1, C1, H * W), lambda b: (b, 0, 0)),
            pl.BlockSpec((2 * C, C1), lambda b: (0, 0)),
            pl.BlockSpec((2 * C, 1), lambda b: (0, 0)),
            pl.BlockSpec((E, C), lambda b: (0, 0)),
            pl.BlockSpec((E, 1), lambda b: (0, 0)),
            pl.BlockSpec((E, 9, C, C), lambda b: (0, 0, 0, 0)),
            pl.BlockSpec((E, C, 1), lambda b: (0, 0, 0)),
            pl.BlockSpec((C2, 3 * C), lambda b: (0, 0)),
            pl.BlockSpec((C2, 1), lambda b: (0, 0)),
        ],
        out_specs=pl.BlockSpec((1, C2, H * W), lambda b: (b, 0, 0)),
        out_shape=jax.ShapeDtypeStruct((B, C2, H * W), jnp.float32),
        scratch_shapes=[pltpu.VMEM((C, NP), jnp.float32)],
    )(xf, W1r, b1c, Wr, brc, Wm, bec, W2r, b2c)

    return out_f.reshape(B, C2, H, W)


# speedup vs baseline: 7.9538x; 1.0028x over previous
import jax
import jax.numpy as jnp
from jax import lax
from jax.experimental import pallas as pl
from jax.experimental.pallas import tpu as pltpu

C1 = 384
C2 = 384
C = 192
E = 4
TOPK = 2
H = 56
W = 56
HP = H + 2
WP = W + 2
NP = HP * WP
NI = (H - 1) * WP + W
OFF0 = WP + 1
STARTS = tuple(i * WP + j for i in range(3) for j in range(3))


def _silu(v):
    return v * jax.nn.sigmoid(v)


def _fused_body(x_ref, W1_ref, b1_ref, Wr_ref, br_ref, Wm_ref, be_ref,
                W2_ref, b2_ref, out_ref, y1p_ref):
    xb = x_ref[0]
    t = jnp.dot(W1_ref[...], xb, preferred_element_type=jnp.float32) + b1_ref[...]
    t = _silu(t)
    y0 = t[:C]
    y1 = t[C:]
    y1p_ref[...] = jnp.zeros((C, NP), jnp.float32)
    for h in range(H):
        y1p_ref[:, (h + 1) * WP + 1:(h + 1) * WP + 1 + W] = y1[:, h * W:(h + 1) * W]
    pooled = jnp.sum(y1, axis=1, keepdims=True) * jnp.float32(1.0 / (H * W))
    logits = jnp.dot(Wr_ref[...], pooled, preferred_element_type=jnp.float32) + br_ref[...]
    m = jnp.max(logits)
    ex = jnp.exp(logits - m)
    p = ex / jnp.sum(ex)
    io = lax.broadcasted_iota(jnp.int32, (E, 1), 0)
    p1 = jnp.max(p)
    i1 = jnp.min(jnp.where(p >= p1, io, E))
    pm = jnp.where(io == i1, -1.0, p)
    p2 = jnp.max(pm)
    i2 = jnp.min(jnp.where(pm >= p2, io, E))
    s = p1 + p2
    w1 = p1 / s
    w2 = p2 / s
    acc = None
    for sidx in range(9):
        wcat = jnp.concatenate([Wm_ref[i1, sidx], Wm_ref[i2, sidx]], axis=0)
        xs = y1p_ref[:, STARTS[sidx]:STARTS[sidx] + NI]
        d = jnp.dot(wcat, xs, preferred_element_type=jnp.float32)
        acc = d if acc is None else acc + d
    e1 = _silu(acc[:C] + be_ref[i1])
    e2 = _silu(acc[C:] + be_ref[i2])
    eo = w1 * e1 + w2 * e2
    eod = jnp.concatenate([eo[:, h * WP:h * WP + W] for h in range(H)], axis=1)
    o = (jnp.dot(W2_ref[:, :C], y0, preferred_element_type=jnp.float32)
         + jnp.dot(W2_ref[:, C:2 * C], y1, preferred_element_type=jnp.float32)
         + jnp.dot(W2_ref[:, 2 * C:], eod, preferred_element_type=jnp.float32)
         + b2_ref[...])
    out_ref[0] = _silu(o)


def kernel(x, W1, b1, Wr, br, We, be, W2, b2):
    B = x.shape[0]
    xf = x.reshape(B, C1, H * W)
    W1r = W1.reshape(2 * C, C1)
    b1c = b1.reshape(2 * C, 1)
    brc = br.reshape(E, 1)
    Wm = We.transpose(0, 3, 4, 1, 2).reshape(E, 9, C, C)
    bec = be.reshape(E, C, 1)
    W2r = W2.reshape(C2, (2 + 1) * C)
    b2c = b2.reshape(C2, 1)

    out_f = pl.pallas_call(
        _fused_body,
        grid=(B,),
        in_specs=[
            pl.BlockSpec((1, C1, H * W), lambda b: (b, 0, 0)),
            pl.BlockSpec((2 * C, C1), lambda b: (0, 0)),
            pl.BlockSpec((2 * C, 1), lambda b: (0, 0)),
            pl.BlockSpec((E, C), lambda b: (0, 0)),
            pl.BlockSpec((E, 1), lambda b: (0, 0)),
            pl.BlockSpec((E, 9, C, C), lambda b: (0, 0, 0, 0)),
            pl.BlockSpec((E, C, 1), lambda b: (0, 0, 0)),
            pl.BlockSpec((C2, 3 * C), lambda b: (0, 0)),
            pl.BlockSpec((C2, 1), lambda b: (0, 0)),
        ],
        out_specs=pl.BlockSpec((1, C2, H * W), lambda b: (b, 0, 0)),
        out_shape=jax.ShapeDtypeStruct((B, C2, H * W), jnp.float32),
        scratch_shapes=[pltpu.VMEM((C, NP), jnp.float32)],
        compiler_params=pltpu.CompilerParams(
            dimension_semantics=("parallel",)),
    )(xf, W1r, b1c, Wr, brc, Wm, bec, W2r, b2c)

    return out_f.reshape(B, C2, H, W)


# speedup vs baseline: 8.4632x; 1.0640x over previous
import jax
import jax.numpy as jnp
from jax import lax
from jax.experimental import pallas as pl
from jax.experimental.pallas import tpu as pltpu

C1 = 384
C2 = 384
C = 192
E = 4
TOPK = 2
H = 56
W = 56
HP = H + 2
WP = W + 2
NP = HP * WP
NI = (H - 1) * WP + W
OFF0 = WP + 1
STARTS = tuple(i * WP + j for i in range(3) for j in range(3))


def _silu(v):
    return v * jax.nn.sigmoid(v)


def _fused_body(x_ref, W1_ref, b1_ref, Wr_ref, br_ref, Wm_ref, be_ref,
                W2_ref, b2_ref, out_ref, y1p_ref):
    xb = x_ref[0].astype(jnp.bfloat16)
    t = jnp.dot(W1_ref[...], xb, preferred_element_type=jnp.float32) + b1_ref[...]
    t = _silu(t)
    y0 = t[:C].astype(jnp.bfloat16)
    y1 = t[C:]
    y1b = y1.astype(jnp.bfloat16)
    y1p_ref[...] = jnp.zeros((C, NP), jnp.bfloat16)
    for h in range(H):
        y1p_ref[:, (h + 1) * WP + 1:(h + 1) * WP + 1 + W] = y1b[:, h * W:(h + 1) * W]
    pooled = jnp.sum(y1, axis=1, keepdims=True) * jnp.float32(1.0 / (H * W))
    logits = jnp.dot(Wr_ref[...], pooled, preferred_element_type=jnp.float32) + br_ref[...]
    m = jnp.max(logits)
    ex = jnp.exp(logits - m)
    p = ex / jnp.sum(ex)
    io = lax.broadcasted_iota(jnp.int32, (E, 1), 0)
    p1 = jnp.max(p)
    i1 = jnp.min(jnp.where(p >= p1, io, E))
    pm = jnp.where(io == i1, -1.0, p)
    p2 = jnp.max(pm)
    i2 = jnp.min(jnp.where(pm >= p2, io, E))
    s = p1 + p2
    w1 = p1 / s
    w2 = p2 / s
    acc = None
    for sidx in range(9):
        wcat = jnp.concatenate([Wm_ref[i1, sidx], Wm_ref[i2, sidx]], axis=0)
        xs = y1p_ref[:, STARTS[sidx]:STARTS[sidx] + NI]
        d = jnp.dot(wcat, xs, preferred_element_type=jnp.float32)
        acc = d if acc is None else acc + d
    e1 = _silu(acc[:C] + be_ref[i1])
    e2 = _silu(acc[C:] + be_ref[i2])
    eo = w1 * e1 + w2 * e2
    eod = jnp.concatenate([eo[:, h * WP:h * WP + W] for h in range(H)], axis=1)
    o = (jnp.dot(W2_ref[:, :C], y0, preferred_element_type=jnp.float32)
         + jnp.dot(W2_ref[:, C:2 * C], y1b, preferred_element_type=jnp.float32)
         + jnp.dot(W2_ref[:, 2 * C:], eod.astype(jnp.bfloat16),
                   preferred_element_type=jnp.float32)
         + b2_ref[...])
    out_ref[0] = _silu(o)


def kernel(x, W1, b1, Wr, br, We, be, W2, b2):
    B = x.shape[0]
    xf = x.reshape(B, C1, H * W)
    W1r = W1.reshape(2 * C, C1).astype(jnp.bfloat16)
    b1c = b1.reshape(2 * C, 1)
    brc = br.reshape(E, 1)
    Wm = We.transpose(0, 3, 4, 1, 2).reshape(E, 9, C, C).astype(jnp.bfloat16)
    bec = be.reshape(E, C, 1)
    W2r = W2.reshape(C2, (2 + 1) * C).astype(jnp.bfloat16)
    b2c = b2.reshape(C2, 1)

    out_f = pl.pallas_call(
        _fused_body,
        grid=(B,),
        in_specs=[
            pl.BlockSpec((1, C1, H * W), lambda b: (b, 0, 0)),
            pl.BlockSpec((2 * C, C1), lambda b: (0, 0)),
            pl.BlockSpec((2 * C, 1), lambda b: (0, 0)),
            pl.BlockSpec((E, C), lambda b: (0, 0)),
            pl.BlockSpec((E, 1), lambda b: (0, 0)),
            pl.BlockSpec((E, 9, C, C), lambda b: (0, 0, 0, 0)),
            pl.BlockSpec((E, C, 1), lambda b: (0, 0, 0)),
            pl.BlockSpec((C2, 3 * C), lambda b: (0, 0)),
            pl.BlockSpec((C2, 1), lambda b: (0, 0)),
        ],
        out_specs=pl.BlockSpec((1, C2, H * W), lambda b: (b, 0, 0)),
        out_shape=jax.ShapeDtypeStruct((B, C2, H * W), jnp.float32),
        scratch_shapes=[pltpu.VMEM((C, NP), jnp.bfloat16)],
        compiler_params=pltpu.CompilerParams(
            dimension_semantics=("parallel",)),
    )(xf, W1r, b1c, Wr, brc, Wm, bec, W2r, b2c)

    return out_f.reshape(B, C2, H, W)
